# baseline (speedup 1.0000x reference)
import jax
import jax.numpy as jnp
from jax import lax
from jax.experimental import pallas as pl
from jax.experimental.pallas import tpu as pltpu

_EPS = 1e-09


def _fused_kernel(z_ref, zt_ref, loss_ref, acc_ref):
    k = pl.program_id(0)

    @pl.when(k == 0)
    def _zero():
        acc_ref[...] = jnp.zeros_like(acc_ref)

    zb = z_ref[...].astype(jnp.bfloat16)
    ztb = zt_ref[...].astype(jnp.bfloat16)
    acc_ref[...] += lax.dot_general(
        zb, ztb,
        dimension_numbers=(((0,), (0,)), ((), ())),
        preferred_element_type=jnp.float32,
    )

    @pl.when(k == pl.num_programs(0) - 1)
    def _epilogue():
        P = acc_ref[...]
        P = (P + P.T) * (0.5 / jnp.sum(P))
        P = jnp.maximum(P, _EPS)
        Pi = jnp.sum(P, axis=1, keepdims=True)
        Pj = jnp.sum(P, axis=0, keepdims=True)
        loss_ref[0, 0] = (jnp.sum(Pi * jnp.log(Pi))
                          + jnp.sum(Pj * jnp.log(Pj))
                          - jnp.sum(P * jnp.log(P)))


def kernel(z, zt):
    n, c = z.shape
    assert zt.shape == (n, c)

    tile_n = 4096
    n_pad = -(-n // tile_n) * tile_n
    if n_pad != n:
        pad = n_pad - n
        z = jnp.pad(z, ((0, pad), (0, 0)))
        zt = jnp.pad(zt, ((0, pad), (0, 0)))
    kt = n_pad // tile_n

    loss = pl.pallas_call(
        _fused_kernel,
        out_shape=jax.ShapeDtypeStruct((1, 1), jnp.float32),
        grid=(kt,),
        in_specs=[
            pl.BlockSpec((tile_n, c), lambda k: (k, 0)),
            pl.BlockSpec((tile_n, c), lambda k: (k, 0)),
        ],
        out_specs=pl.BlockSpec(memory_space=pltpu.MemorySpace.SMEM),
        scratch_shapes=[pltpu.VMEM((c, c), jnp.float32)],
        compiler_params=pltpu.CompilerParams(
            dimension_semantics=("arbitrary",),
            vmem_limit_bytes=56 * 1024 * 1024,
        ),
        cost_estimate=pl.CostEstimate(
            flops=2 * n_pad * c * c,
            transcendentals=c * c + 2 * c,
            bytes_accessed=2 * n_pad * c * 4 + 4,
        ),
    )(z, zt)
    return loss[0, 0]


# speedup vs baseline: 1.0956x; 1.0956x over previous
import jax
import jax.numpy as jnp
from jax import lax
from jax.experimental import pallas as pl
from jax.experimental.pallas import tpu as pltpu

_EPS = 1e-09


def _iic_fused_kernel(z_ref, zt_ref, loss_ref, acc_ref):
    k = pl.program_id(0)

    @pl.when(k == 0)
    def _zero():
        acc_ref[...] = jnp.zeros_like(acc_ref)

    zb = z_ref[...].astype(jnp.bfloat16)
    ztb = zt_ref[...].astype(jnp.bfloat16)
    acc_ref[...] += lax.dot_general(
        zb, ztb,
        dimension_numbers=(((0,), (0,)), ((), ())),
        preferred_element_type=jnp.float32,
    )

    @pl.when(k == pl.num_programs(0) - 1)
    def _epilogue():
        P = acc_ref[...]
        P = (P + P.T) * (0.5 / jnp.sum(P))
        P = jnp.maximum(P, _EPS)
        Pi = jnp.sum(P, axis=1, keepdims=True)
        Pj = jnp.sum(P, axis=0, keepdims=True)
        loss_ref[0, 0] = (jnp.sum(Pi * jnp.log(Pi))
                          + jnp.sum(Pj * jnp.log(Pj))
                          - jnp.sum(P * jnp.log(P)))


def kernel(z, zt):
    n, c = z.shape
    assert zt.shape == (n, c)

    tile_n = 8192
    n_pad = -(-n // tile_n) * tile_n
    if n_pad != n:
        pad = n_pad - n
        z = jnp.pad(z, ((0, pad), (0, 0)))
        zt = jnp.pad(zt, ((0, pad), (0, 0)))
    kt = n_pad // tile_n

    loss = pl.pallas_call(
        _iic_fused_kernel,
        out_shape=jax.ShapeDtypeStruct((1, 1), jnp.float32),
        grid=(kt,),
        in_specs=[
            pl.BlockSpec((tile_n, c), lambda k: (k, 0)),
            pl.BlockSpec((tile_n, c), lambda k: (k, 0)),
        ],
        out_specs=pl.BlockSpec(memory_space=pltpu.MemorySpace.SMEM),
        scratch_shapes=[pltpu.VMEM((c, c), jnp.float32)],
        compiler_params=pltpu.CompilerParams(
            dimension_semantics=("arbitrary",),
            vmem_limit_bytes=56 * 1024 * 1024,
        ),
        cost_estimate=pl.CostEstimate(
            flops=2 * n_pad * c * c,
            transcendentals=c * c + 2 * c,
            bytes_accessed=2 * n_pad * c * 4 + 4,
        ),
    )(z, zt)
    return loss[0, 0]
